# baseline (speedup 1.0000x reference)
import jax
import jax.numpy as jnp
from jax import lax
from jax.experimental import pallas as pl
from jax.experimental.pallas import tpu as pltpu
from jax.experimental.pallas import tpu_sc as plsc

BATCH = 4
TOKENS = 200
DIM = 4096
CHUNK = 8
ROW_CHUNKS = 25
WPR = 8


def _gather_body(idx_hbm, table_hbm, out_hbm, idx_v, rows_v, gsem, wsem):
    wid = lax.axis_index("s") * 2 + lax.axis_index("c")
    b = wid // WPR
    j = wid % WPR
    s = jnp.where(j == 0, 0, 3 * j + 1)
    cnt = jnp.where(j == 0, 4, 3)

    pltpu.sync_copy(idx_hbm, idx_v)

    def step(k, _):
        off = pl.multiple_of((s + k) * CHUNK, CHUNK)
        g = pltpu.make_async_copy(
            table_hbm.at[idx_v.at[b, pl.ds(off, CHUNK)]], rows_v, gsem)
        g.start()
        g.wait()
        w = pltpu.make_async_copy(
            rows_v, out_hbm.at[b, pl.ds(off, CHUNK)], wsem)
        w.start()
        w.wait()
        return 0

    lax.fori_loop(0, cnt, step, 0)


@jax.jit
def kernel(indices, embedding_table):
    mesh = plsc.VectorSubcoreMesh(core_axis_name="c", subcore_axis_name="s")
    return pl.kernel(
        _gather_body,
        mesh=mesh,
        out_type=jax.ShapeDtypeStruct((BATCH, TOKENS, DIM), jnp.float32),
        scratch_types=[
            pltpu.VMEM((BATCH, TOKENS), jnp.int32),
            pltpu.VMEM((CHUNK, DIM), jnp.float32),
            pltpu.SemaphoreType.DMA,
            pltpu.SemaphoreType.DMA,
        ],
    )(indices.astype(jnp.int32), embedding_table)
